# Initial kernel scaffold; baseline (speedup 1.0000x reference)
import jax
import jax.numpy as jnp
from jax.experimental import pallas as pl


def kernel(x, edge_index, edge_attr, edge_time, w_t, b_t, mem, W_msg, b_msg, Wz, Uz, bz, Wr, Ur, br, Wh, Uh, bh, W_emb, b_emb, W_out, b_out, W_cls, b_cls):
    raise NotImplementedError("write your pallas kernel here")


# speedup vs baseline: 2.2962x; 2.2962x over previous
import functools

import jax
import jax.numpy as jnp
from jax import lax
from jax.experimental import pallas as pl
from jax.experimental.pallas import tpu as pltpu
from jax.experimental.pallas import tpu_sc as plsc

N = 10000
E = 320000
D_FEAT = 128
D_EDGE = 16
HID = 64
TENC = 32
OUT = 64
NACT = 50

NC, NS, LANES = 2, 16, 16
NW = NC * NS
CHUNK = 128
NCHUNKS = E // CHUNK
FULL = NCHUNKS // NW
REM = NCHUNKS - FULL * NW
AGG_W = 128
TAB_W = 2 * HID
ROWS_PER_TILE = 624
TAIL_ROW0 = NS * ROWS_PER_TILE
TAIL_ROWS = N - TAIL_ROW0
ZCHUNKS = ((0, 128), (128, 128), (256, 128), (384, 128), (512, 112))

_mesh = plsc.VectorSubcoreMesh(
    core_axis_name="c", subcore_axis_name="s", num_cores=NC, num_subcores=NS
)


def _edge_pre_body(time_ref, attr_ref, wt_ref, bt_ref, w34_ref, bm_ref, c_ref):
    tenc = jnp.cos(time_ref[...] * wt_ref[...] + bt_ref[...])
    ta = jnp.concatenate([attr_ref[...], tenc], axis=1)
    c_ref[...] = (
        jnp.dot(ta, w34_ref[...], preferred_element_type=jnp.float32)
        + bm_ref[...]
    )


def _edge_pre(edge_time, edge_attr, w_t, b_t, w34, b_msg):
    eb = 8000
    return pl.pallas_call(
        _edge_pre_body,
        grid=(E // eb,),
        in_specs=[
            pl.BlockSpec((eb, 1), lambda i: (i, 0)),
            pl.BlockSpec((eb, D_EDGE), lambda i: (i, 0)),
            pl.BlockSpec((1, TENC), lambda i: (0, 0)),
            pl.BlockSpec((1, TENC), lambda i: (0, 0)),
            pl.BlockSpec((D_EDGE + TENC, HID), lambda i: (0, 0)),
            pl.BlockSpec((1, HID), lambda i: (0, 0)),
        ],
        out_specs=pl.BlockSpec((eb, HID), lambda i: (i, 0)),
        out_shape=jax.ShapeDtypeStruct((E, HID), jnp.float32),
    )(
        edge_time.reshape(E, 1),
        edge_attr,
        w_t.reshape(1, TENC),
        b_t.reshape(1, TENC),
        w34,
        b_msg.reshape(1, HID),
    )


def _ab_body(mem_ref, w12_ref, ab_ref):
    ab_ref[...] = jnp.dot(
        mem_ref[...], w12_ref[...], preferred_element_type=jnp.float32
    )


def _ab(mem, w12):
    return pl.pallas_call(
        _ab_body,
        out_shape=jax.ShapeDtypeStruct((N, TAB_W), jnp.float32),
    )(mem, w12)


def _node_body(
    aggp_ref, mem_ref, x_ref, wz_ref, uz_ref, bz_ref, wr_ref, ur_ref, br_ref,
    wh_ref, uh_ref, bh_ref, wex_ref, wem_ref, bemb_ref, wout_ref, bout_ref,
    wcls_ref, bcls_ref, cls_ref,
):
    dot = functools.partial(jnp.dot, preferred_element_type=jnp.float32)
    ap = aggp_ref[...]
    ssum = ap[0] + ap[1]
    cnt = jnp.maximum(ssum[:, HID:HID + 1], 1.0)
    agg = ssum[:, :HID] / cnt
    mem = mem_ref[...]
    z = jax.nn.sigmoid(dot(agg, wz_ref[...]) + dot(mem, uz_ref[...]) + bz_ref[...])
    r = jax.nn.sigmoid(dot(agg, wr_ref[...]) + dot(mem, ur_ref[...]) + br_ref[...])
    h = jnp.tanh(dot(agg, wh_ref[...]) + dot(r * mem, uh_ref[...]) + bh_ref[...])
    mem_new = (1.0 - z) * mem + z * h
    emb = jax.nn.relu(
        dot(x_ref[...], wex_ref[...]) + dot(mem_new, wem_ref[...]) + bemb_ref[...]
    )
    node_out = dot(emb, wout_ref[...]) + bout_ref[...]
    cls_ref[...] = dot(node_out, wcls_ref[...]) + bcls_ref[...]


def _node(aggp, mem, x, wz, uz, bz, wr, ur, br, wh, uh, bh,
          wex, wem, bemb, wout, bout, wcls_pad, bcls_pad):
    nb = 2000
    full = lambda shape: pl.BlockSpec(shape, lambda i: tuple(0 for _ in shape))
    return pl.pallas_call(
        _node_body,
        grid=(N // nb,),
        in_specs=[
            pl.BlockSpec((NC, nb, AGG_W), lambda i: (0, i, 0)),
            pl.BlockSpec((nb, HID), lambda i: (i, 0)),
            pl.BlockSpec((nb, D_FEAT), lambda i: (i, 0)),
            full((HID, HID)), full((HID, HID)), full((1, HID)),
            full((HID, HID)), full((HID, HID)), full((1, HID)),
            full((HID, HID)), full((HID, HID)), full((1, HID)),
            full((D_FEAT, HID)), full((HID, HID)), full((1, HID)),
            full((HID, OUT)), full((1, OUT)),
            full((OUT, AGG_W)), full((1, AGG_W)),
        ],
        out_specs=pl.BlockSpec((nb, AGG_W), lambda i: (i, 0)),
        out_shape=jax.ShapeDtypeStruct((N, AGG_W), jnp.float32),
    )(
        aggp, mem, x,
        wz, uz, bz.reshape(1, HID),
        wr, ur, br.reshape(1, HID),
        wh, uh, bh.reshape(1, HID),
        wex, wem, bemb.reshape(1, HID),
        wout, bout.reshape(1, OUT),
        wcls_pad, bcls_pad.reshape(1, AGG_W),
    )


@functools.partial(
    pl.kernel,
    mesh=_mesh,
    out_type=jax.ShapeDtypeStruct((NC, N, AGG_W), jnp.float32),
    scratch_types=[
        pltpu.VMEM((1, CHUNK), jnp.int32),
        pltpu.VMEM((1, CHUNK), jnp.int32),
        pltpu.VMEM((CHUNK, TAB_W), jnp.float32),
        pltpu.VMEM((CHUNK, HID), jnp.float32),
        pltpu.VMEM((CHUNK, AGG_W), jnp.float32),
        pltpu.VMEM_SHARED((N, AGG_W), jnp.float32),
        pltpu.SemaphoreType.DMA,
        pltpu.SemaphoreType.DMA,
        pltpu.SemaphoreType.DMA,
    ],
)
def _msg_kernel(src_hbm, dst_hbm, ab_hbm, c_hbm, out_hbm,
                idx_s, idx_d, buf_a, buf_c, buf_m, agg_sh,
                sem_a, sem_b, sem_c):
    cid = lax.axis_index("c")
    sid = lax.axis_index("s")
    wid = cid * NS + sid

    zeros = jnp.zeros((LANES,), jnp.float32)

    def zero_row(i, _):
        for j in range(AGG_W // LANES):
            buf_m[i, pl.ds(j * LANES, LANES)] = zeros
        return 0

    lax.fori_loop(0, CHUNK, zero_row, 0)

    for off, nr in ZCHUNKS:
        pltpu.sync_copy(
            buf_m.at[pl.ds(0, nr), :],
            agg_sh.at[pl.ds(sid * ROWS_PER_TILE + off, nr), :],
        )

    @pl.when(sid == NS - 1)
    def _():
        pltpu.sync_copy(
            buf_m.at[pl.ds(0, TAIL_ROWS), :],
            agg_sh.at[pl.ds(TAIL_ROW0, TAIL_ROWS), :],
        )

    pat = jnp.where(lax.iota(jnp.int32, LANES) == 0, 1.0, 0.0)

    plsc.subcore_barrier()

    def do_chunk(ci):
        base = ci * CHUNK
        pltpu.sync_copy(src_hbm.at[pl.ds(base, CHUNK)], idx_s.at[0])
        pltpu.sync_copy(dst_hbm.at[pl.ds(base, CHUNK)], idx_d.at[0])
        cp_a = pltpu.async_copy(ab_hbm.at[idx_s.at[0]], buf_a, sem_a)
        cp_b = pltpu.async_copy(ab_hbm.at[idx_d.at[0]], buf_m, sem_b)
        cp_c = pltpu.async_copy(c_hbm.at[pl.ds(base, CHUNK), :], buf_c, sem_c)
        cp_a.wait()
        cp_b.wait()
        cp_c.wait()

        def row(i, _):
            for j in range(HID // LANES):
                sl = pl.ds(j * LANES, LANES)
                v = buf_a[i, sl] + buf_m[i, pl.ds(HID + j * LANES, LANES)] \
                    + buf_c[i, sl]
                buf_m[i, sl] = jnp.maximum(v, 0.0)
            buf_m[i, pl.ds(HID, LANES)] = pat
            return 0

        lax.fori_loop(0, CHUNK, row, 0)
        pltpu.sync_copy(buf_m, agg_sh.at[idx_d.at[0]], add=True)

    def round_(g, _):
        do_chunk(wid + NW * g)
        return 0

    lax.fori_loop(0, FULL, round_, 0)

    @pl.when(wid < REM)
    def _():
        do_chunk(FULL * NW + wid)

    plsc.subcore_barrier()

    for off, nr in ZCHUNKS:
        r0 = sid * ROWS_PER_TILE + off
        pltpu.sync_copy(
            agg_sh.at[pl.ds(r0, nr), :],
            out_hbm.at[cid, pl.ds(r0, nr), :],
        )

    @pl.when(sid == NS - 1)
    def _():
        pltpu.sync_copy(
            agg_sh.at[pl.ds(TAIL_ROW0, TAIL_ROWS), :],
            out_hbm.at[cid, pl.ds(TAIL_ROW0, TAIL_ROWS), :],
        )


@functools.partial(
    pl.kernel,
    mesh=_mesh,
    out_type=jax.ShapeDtypeStruct((E, AGG_W), jnp.float32),
    scratch_types=[
        pltpu.VMEM((1, CHUNK), jnp.int32),
        pltpu.VMEM((CHUNK, AGG_W), jnp.float32),
        pltpu.SemaphoreType.DMA,
    ],
)
def _cls_kernel(dst_hbm, cls_hbm, out_hbm, idx_d, rows, sem):
    cid = lax.axis_index("c")
    sid = lax.axis_index("s")
    wid = cid * NS + sid

    def do_chunk(ci):
        base = ci * CHUNK
        pltpu.sync_copy(dst_hbm.at[pl.ds(base, CHUNK)], idx_d.at[0])
        pltpu.async_copy(cls_hbm.at[idx_d.at[0]], rows, sem).wait()
        pltpu.sync_copy(rows, out_hbm.at[pl.ds(base, CHUNK), :])

    def round_(g, _):
        do_chunk(wid + NW * g)
        return 0

    lax.fori_loop(0, FULL, round_, 0)

    @pl.when(wid < REM)
    def _():
        do_chunk(FULL * NW + wid)


def kernel(x, edge_index, edge_attr, edge_time, w_t, b_t, mem, W_msg, b_msg,
           Wz, Uz, bz, Wr, Ur, br, Wh, Uh, bh, W_emb, b_emb, W_out, b_out,
           W_cls, b_cls):
    src = edge_index[0]
    dst = edge_index[1]

    c_e = _edge_pre(edge_time, edge_attr, w_t, b_t, W_msg[2 * HID:], b_msg)
    w12 = jnp.concatenate([W_msg[:HID], W_msg[HID:2 * HID]], axis=1)
    ab_tab = _ab(mem, w12)

    aggp = _msg_kernel(src, dst, ab_tab, c_e)

    wcls_pad = jnp.pad(W_cls, ((0, 0), (0, AGG_W - NACT)))
    bcls_pad = jnp.pad(b_cls, (0, AGG_W - NACT))
    cls_all = _node(
        aggp, mem, x, Wz, Uz, bz, Wr, Ur, br, Wh, Uh, bh,
        W_emb[:D_FEAT], W_emb[D_FEAT:], b_emb, W_out, b_out,
        wcls_pad, bcls_pad,
    )

    return _cls_kernel(dst, cls_all)[:, :NACT]


# speedup vs baseline: 5.2662x; 2.2934x over previous
import functools

import jax
import jax.numpy as jnp
from jax import lax
from jax.experimental import pallas as pl
from jax.experimental.pallas import tpu as pltpu
from jax.experimental.pallas import tpu_sc as plsc

N = 10000
E = 320000
D_FEAT = 128
D_EDGE = 16
HID = 64
TENC = 32
OUT = 64
NACT = 50

NC, NS, LANES = 2, 16, 16
NW = NC * NS
CHUNK = 64
NCHUNKS = E // CHUNK
FULL = NCHUNKS // NW
REM = NCHUNKS - FULL * NW
CHUNK2 = 128
NCHUNKS2 = E // CHUNK2
FULL2 = NCHUNKS2 // NW
REM2 = NCHUNKS2 - FULL2 * NW
AGG_W = 128
TAB_W = 2 * HID
ROWS_PER_TILE = 624
TAIL_ROW0 = NS * ROWS_PER_TILE
TAIL_ROWS = N - TAIL_ROW0
ZCHUNKS = tuple((k * CHUNK, CHUNK) for k in range(ROWS_PER_TILE // CHUNK)) + (
    (ROWS_PER_TILE - ROWS_PER_TILE % CHUNK, ROWS_PER_TILE % CHUNK),
) if ROWS_PER_TILE % CHUNK else tuple(
    (k * CHUNK, CHUNK) for k in range(ROWS_PER_TILE // CHUNK))

_mesh = plsc.VectorSubcoreMesh(
    core_axis_name="c", subcore_axis_name="s", num_cores=NC, num_subcores=NS
)


_COS_POLY = (0.99999999994488, -0.4999999985174111, 0.04166666348459491,
             -0.0013888863060936606, 2.4800554073532125e-05,
             -2.753480677587447e-07, 2.0603612843617114e-09,
             -9.722556093549883e-12)
_INV_2PI = 0.15915494309189535
_MAGIC = 12582912.0
_PI_HI = 6.28125
_PI_LO = 2.0 * 3.141592653589793 - 6.28125


def _fast_cos(x):
    k = (x * _INV_2PI + _MAGIC) - _MAGIC
    r = (x - k * _PI_HI) - k * _PI_LO
    u = r * r
    p = jnp.float32(_COS_POLY[-1])
    for c in _COS_POLY[-2::-1]:
        p = p * u + jnp.float32(c)
    return p


def _edge_pre_body(time_ref, attrt_ref, wt_ref, bt_ref, w4_ref, w3_ref,
                   bm_ref, c_ref):
    t = time_ref[...]
    rows = t.shape[0]
    trow = jnp.concatenate([t[i:i + 1, :] for i in range(rows)], axis=1)
    z = wt_ref[...] * trow + bt_ref[...]
    tenc_t = _fast_cos(z)
    contract0 = (((0,), (0,)), ((), ()))
    c_ref[...] = (
        lax.dot_general(tenc_t, w4_ref[...], contract0,
                        preferred_element_type=jnp.float32)
        + lax.dot_general(attrt_ref[...], w3_ref[...], contract0,
                          preferred_element_type=jnp.float32)
        + bm_ref[...]
    )


def _edge_pre(edge_time, edge_attr, w_t, b_t, w34, b_msg):
    eb = 8192
    erows = eb // 128
    return pl.pallas_call(
        _edge_pre_body,
        grid=(pl.cdiv(E, eb),),
        in_specs=[
            pl.BlockSpec((erows, 128), lambda i: (i, 0)),
            pl.BlockSpec((D_EDGE, eb), lambda i: (0, i)),
            pl.BlockSpec((TENC, 1), lambda i: (0, 0)),
            pl.BlockSpec((TENC, 1), lambda i: (0, 0)),
            pl.BlockSpec((TENC, HID), lambda i: (0, 0)),
            pl.BlockSpec((D_EDGE, HID), lambda i: (0, 0)),
            pl.BlockSpec((1, HID), lambda i: (0, 0)),
        ],
        out_specs=pl.BlockSpec((eb, HID), lambda i: (i, 0)),
        out_shape=jax.ShapeDtypeStruct((E, HID), jnp.float32),
    )(
        edge_time.reshape(E // 128, 128),
        edge_attr.T,
        w_t.reshape(TENC, 1),
        b_t.reshape(TENC, 1),
        w34[D_EDGE:],
        w34[:D_EDGE],
        b_msg.reshape(1, HID),
    )


def _ab_body(mem_ref, w12_ref, ab_ref):
    ab_ref[...] = jnp.dot(
        mem_ref[...], w12_ref[...], preferred_element_type=jnp.float32
    )


def _ab(mem, w12):
    return pl.pallas_call(
        _ab_body,
        out_shape=jax.ShapeDtypeStruct((N, TAB_W), jnp.float32),
    )(mem, w12)


def _node_body(
    aggp_ref, mem_ref, x_ref, wz_ref, uz_ref, bz_ref, wr_ref, ur_ref, br_ref,
    wh_ref, uh_ref, bh_ref, wex_ref, wem_ref, bemb_ref, wout_ref, bout_ref,
    wcls_ref, bcls_ref, cls_ref,
):
    dot = functools.partial(jnp.dot, preferred_element_type=jnp.float32)
    ap = aggp_ref[...]
    ssum = ap[0] + ap[1]
    cnt = jnp.maximum(ssum[:, HID:HID + 1], 1.0)
    agg = ssum[:, :HID] / cnt
    mem = mem_ref[...]
    z = jax.nn.sigmoid(dot(agg, wz_ref[...]) + dot(mem, uz_ref[...]) + bz_ref[...])
    r = jax.nn.sigmoid(dot(agg, wr_ref[...]) + dot(mem, ur_ref[...]) + br_ref[...])
    h = jnp.tanh(dot(agg, wh_ref[...]) + dot(r * mem, uh_ref[...]) + bh_ref[...])
    mem_new = (1.0 - z) * mem + z * h
    emb = jax.nn.relu(
        dot(x_ref[...], wex_ref[...]) + dot(mem_new, wem_ref[...]) + bemb_ref[...]
    )
    node_out = dot(emb, wout_ref[...]) + bout_ref[...]
    cls_ref[...] = dot(node_out, wcls_ref[...]) + bcls_ref[...]


def _node(aggp, mem, x, wz, uz, bz, wr, ur, br, wh, uh, bh,
          wex, wem, bemb, wout, bout, wcls_pad, bcls_pad):
    nb = 2000
    full = lambda shape: pl.BlockSpec(shape, lambda i: tuple(0 for _ in shape))
    return pl.pallas_call(
        _node_body,
        grid=(N // nb,),
        in_specs=[
            pl.BlockSpec((NC, nb, AGG_W), lambda i: (0, i, 0)),
            pl.BlockSpec((nb, HID), lambda i: (i, 0)),
            pl.BlockSpec((nb, D_FEAT), lambda i: (i, 0)),
            full((HID, HID)), full((HID, HID)), full((1, HID)),
            full((HID, HID)), full((HID, HID)), full((1, HID)),
            full((HID, HID)), full((HID, HID)), full((1, HID)),
            full((D_FEAT, HID)), full((HID, HID)), full((1, HID)),
            full((HID, OUT)), full((1, OUT)),
            full((OUT, AGG_W)), full((1, AGG_W)),
        ],
        out_specs=pl.BlockSpec((nb, AGG_W), lambda i: (i, 0)),
        out_shape=jax.ShapeDtypeStruct((N, AGG_W), jnp.float32),
    )(
        aggp, mem, x,
        wz, uz, bz.reshape(1, HID),
        wr, ur, br.reshape(1, HID),
        wh, uh, bh.reshape(1, HID),
        wex, wem, bemb.reshape(1, HID),
        wout, bout.reshape(1, OUT),
        wcls_pad, bcls_pad.reshape(1, AGG_W),
    )


@functools.partial(
    pl.kernel,
    mesh=_mesh,
    out_type=jax.ShapeDtypeStruct((NC, N, AGG_W), jnp.float32),
    scratch_types=[
        pltpu.VMEM((1, CHUNK), jnp.int32),
        pltpu.VMEM((1, CHUNK), jnp.int32),
        pltpu.VMEM((CHUNK, TAB_W), jnp.float32),
        pltpu.VMEM((CHUNK, HID), jnp.float32),
        pltpu.VMEM((CHUNK, AGG_W), jnp.float32),
        pltpu.VMEM((1, CHUNK), jnp.int32),
        pltpu.VMEM((1, CHUNK), jnp.int32),
        pltpu.VMEM((CHUNK, TAB_W), jnp.float32),
        pltpu.VMEM((CHUNK, HID), jnp.float32),
        pltpu.VMEM((CHUNK, AGG_W), jnp.float32),
        pltpu.VMEM_SHARED((N, AGG_W), jnp.float32),
        pltpu.SemaphoreType.DMA,
        pltpu.SemaphoreType.DMA,
    ],
)
def _msg_kernel(src_hbm, dst_hbm, ab_hbm, c_hbm, out_hbm,
                idx_s0, idx_d0, buf_a0, buf_c0, buf_m0,
                idx_s1, idx_d1, buf_a1, buf_c1, buf_m1,
                agg_sh, sem0, sem1):
    cid = lax.axis_index("c")
    sid = lax.axis_index("s")
    wid = cid * NS + sid
    slots = (
        (idx_s0, idx_d0, buf_a0, buf_c0, buf_m0, sem0),
        (idx_s1, idx_d1, buf_a1, buf_c1, buf_m1, sem1),
    )

    zeros = jnp.zeros((LANES,), jnp.float32)

    @plsc.parallel_loop(0, CHUNK, unroll=4)
    def _(i):
        for j in range(AGG_W // LANES):
            buf_m0[i, pl.ds(j * LANES, LANES)] = zeros

    for off, nr in ZCHUNKS:
        pltpu.sync_copy(
            buf_m0.at[pl.ds(0, nr), :],
            agg_sh.at[pl.ds(sid * ROWS_PER_TILE + off, nr), :],
        )

    @pl.when(sid == NS - 1)
    def _():
        pltpu.sync_copy(
            buf_m0.at[pl.ds(0, TAIL_ROWS), :],
            agg_sh.at[pl.ds(TAIL_ROW0, TAIL_ROWS), :],
        )

    pat = jnp.where(lax.iota(jnp.int32, LANES) == 0, 1.0, 0.0)

    plsc.subcore_barrier()

    start = wid * FULL + jnp.minimum(wid, REM)
    n_extra = wid < REM

    def fetch(slot, ci):
        idx_s, idx_d, buf_a, buf_c, buf_m, sem = slots[slot]
        base = ci * CHUNK
        pltpu.sync_copy(src_hbm.at[pl.ds(base, CHUNK)], idx_s.at[0])
        pltpu.sync_copy(dst_hbm.at[pl.ds(base, CHUNK)], idx_d.at[0])
        pltpu.async_copy(ab_hbm.at[idx_s.at[0]], buf_a, sem)
        pltpu.async_copy(ab_hbm.at[idx_d.at[0]], buf_m, sem)
        pltpu.async_copy(c_hbm.at[pl.ds(base, CHUNK), :], buf_c, sem)

    def process(slot):
        idx_s, idx_d, buf_a, buf_c, buf_m, sem = slots[slot]
        pltpu.make_async_copy(ab_hbm.at[idx_s.at[0]], buf_a, sem).wait()
        pltpu.make_async_copy(ab_hbm.at[idx_d.at[0]], buf_m, sem).wait()
        pltpu.make_async_copy(c_hbm.at[pl.ds(0, CHUNK), :], buf_c, sem).wait()

        @plsc.parallel_loop(0, CHUNK, unroll=4)
        def _(i):
            for j in range(HID // LANES):
                sl = pl.ds(j * LANES, LANES)
                v = buf_a[i, sl] + buf_m[i, pl.ds(HID + j * LANES, LANES)] \
                    + buf_c[i, sl]
                buf_m[i, sl] = jnp.maximum(v, 0.0)

        @plsc.parallel_loop(0, CHUNK, unroll=8)
        def _(i):
            buf_m[i, pl.ds(HID, LANES)] = pat

        pltpu.sync_copy(buf_m, agg_sh.at[idx_d.at[0]], add=True)

    fetch(0, start)

    def pair(p, _):
        fetch(1, start + 2 * p + 1)
        process(0)

        @pl.when(p < FULL // 2 - 1)
        def _():
            fetch(0, start + 2 * p + 2)

        process(1)
        return 0

    lax.fori_loop(0, FULL // 2, pair, 0)

    @pl.when(n_extra)
    def _():
        fetch(0, start + FULL)
        process(0)

    plsc.subcore_barrier()

    for off, nr in ZCHUNKS:
        r0 = sid * ROWS_PER_TILE + off
        pltpu.sync_copy(
            agg_sh.at[pl.ds(r0, nr), :],
            out_hbm.at[cid, pl.ds(r0, nr), :],
        )

    @pl.when(sid == NS - 1)
    def _():
        pltpu.sync_copy(
            agg_sh.at[pl.ds(TAIL_ROW0, TAIL_ROWS), :],
            out_hbm.at[cid, pl.ds(TAIL_ROW0, TAIL_ROWS), :],
        )


@functools.partial(
    pl.kernel,
    mesh=_mesh,
    out_type=jax.ShapeDtypeStruct((E, AGG_W), jnp.float32),
    scratch_types=[
        pltpu.VMEM((1, CHUNK2), jnp.int32),
        pltpu.VMEM((CHUNK2, AGG_W), jnp.float32),
        pltpu.VMEM((1, CHUNK2), jnp.int32),
        pltpu.VMEM((CHUNK2, AGG_W), jnp.float32),
        pltpu.SemaphoreType.DMA,
        pltpu.SemaphoreType.DMA,
    ],
)
def _cls_kernel(dst_hbm, cls_hbm, out_hbm,
                idx0, rows0, idx1, rows1, sem0, sem1):
    cid = lax.axis_index("c")
    sid = lax.axis_index("s")
    wid = cid * NS + sid
    slots = ((idx0, rows0, sem0), (idx1, rows1, sem1))

    start = wid * FULL2 + jnp.minimum(wid, REM2)
    n_extra = wid < REM2

    def fetch(slot, ci):
        idx, rows, sem = slots[slot]
        base = ci * CHUNK2
        pltpu.sync_copy(dst_hbm.at[pl.ds(base, CHUNK2)], idx.at[0])
        pltpu.async_copy(cls_hbm.at[idx.at[0]], rows, sem)

    def process(slot, ci):
        idx, rows, sem = slots[slot]
        base = ci * CHUNK2
        pltpu.make_async_copy(cls_hbm.at[idx.at[0]], rows, sem).wait()
        pltpu.sync_copy(rows, out_hbm.at[pl.ds(base, CHUNK2), :])

    fetch(0, start)

    def pair(p, _):
        fetch(1, start + 2 * p + 1)
        process(0, start + 2 * p)

        @pl.when(p < FULL2 // 2 - 1)
        def _():
            fetch(0, start + 2 * p + 2)

        process(1, start + 2 * p + 1)
        return 0

    lax.fori_loop(0, FULL2 // 2, pair, 0)

    @pl.when(n_extra)
    def _():
        fetch(0, start + FULL2)
        process(0, start + FULL2)


def kernel(x, edge_index, edge_attr, edge_time, w_t, b_t, mem, W_msg, b_msg,
           Wz, Uz, bz, Wr, Ur, br, Wh, Uh, bh, W_emb, b_emb, W_out, b_out,
           W_cls, b_cls):
    src = edge_index[0]
    dst = edge_index[1]

    c_e = _edge_pre(edge_time, edge_attr, w_t, b_t, W_msg[2 * HID:], b_msg)
    w12 = jnp.concatenate([W_msg[:HID], W_msg[HID:2 * HID]], axis=1)
    ab_tab = _ab(mem, w12)

    aggp = _msg_kernel(src, dst, ab_tab, c_e)

    wcls_pad = jnp.pad(W_cls, ((0, 0), (0, AGG_W - NACT)))
    bcls_pad = jnp.pad(b_cls, (0, AGG_W - NACT))
    cls_all = _node(
        aggp, mem, x, Wz, Uz, bz, Wr, Ur, br, Wh, Uh, bh,
        W_emb[:D_FEAT], W_emb[D_FEAT:], b_emb, W_out, b_out,
        wcls_pad, bcls_pad,
    )

    return _cls_kernel(dst, cls_all)[:, :NACT]
